# Initial kernel scaffold; baseline (speedup 1.0000x reference)
import jax
import jax.numpy as jnp
from jax.experimental import pallas as pl


def kernel(x, edge_index, W0, b0, W1, b1, W2, b2, Wout, bout):
    raise NotImplementedError("write your pallas kernel here")


# speedup vs baseline: 4.8627x; 4.8627x over previous
import functools

import jax
import jax.numpy as jnp
from jax import lax
from jax.experimental import pallas as pl
from jax.experimental.pallas import tpu as pltpu
from jax.experimental.pallas import tpu_sc as plsc

NC = 2
NS = 16
NW = NC * NS
CHUNK = 80


ROWBLK = 80


def _seg_sum_body(n_nodes, feat, edges_per_w, h_hbm, src_hbm, dst_hbm,
                  out_hbm, agg_sh, sidx_v, didx_v, rows_v, zbuf_v, sem):
    c = lax.axis_index("c")
    s = lax.axis_index("s")
    wid = c * NS + s

    nblk = n_nodes // ROWBLK
    blk_iters = (nblk + NS - 1) // NS

    def zero_row(r, _):
        for j in range(feat // 16):
            zbuf_v[r, pl.ds(j * 16, 16)] = jnp.zeros((16,), jnp.float32)
        return 0
    lax.fori_loop(0, ROWBLK, zero_row, 0)

    def zero_blk(k, _):
        b = s + k * NS

        @pl.when(b < nblk)
        def _():
            pltpu.sync_copy(zbuf_v, agg_sh.at[pl.ds(b * ROWBLK, ROWBLK)])
        return 0
    lax.fori_loop(0, blk_iters, zero_blk, 0)
    plsc.subcore_barrier()

    base = wid * edges_per_w

    def edge_chunk(k, _):
        off = base + k * CHUNK
        pltpu.sync_copy(src_hbm.at[pl.ds(off, CHUNK)], sidx_v)
        pltpu.sync_copy(dst_hbm.at[pl.ds(off, CHUNK)], didx_v)
        pltpu.async_copy(h_hbm.at[sidx_v], rows_v, sem).wait()
        pltpu.sync_copy(rows_v, agg_sh.at[didx_v], add=True)
        return 0
    lax.fori_loop(0, edges_per_w // CHUNK, edge_chunk, 0)
    plsc.subcore_barrier()

    def wb_blk(k, _):
        b = s + k * NS

        @pl.when(b < nblk)
        def _():
            pltpu.sync_copy(agg_sh.at[pl.ds(b * ROWBLK, ROWBLK)],
                            out_hbm.at[c, pl.ds(b * ROWBLK, ROWBLK)])
        return 0
    lax.fori_loop(0, blk_iters, wb_blk, 0)


def _segment_sum_sc(h, src, dst):
    n_nodes, feat = h.shape
    e = src.shape[0]
    assert e % NW == 0
    edges_per_w = e // NW
    assert edges_per_w % CHUNK == 0
    mesh = plsc.VectorSubcoreMesh(core_axis_name="c", subcore_axis_name="s")
    body = functools.partial(_seg_sum_body, n_nodes, feat, edges_per_w)
    return pl.kernel(
        body,
        out_type=jax.ShapeDtypeStruct((NC, n_nodes, feat), jnp.float32),
        mesh=mesh,
        scratch_types=[
            pltpu.VMEM_SHARED((n_nodes, feat), jnp.float32),
            pltpu.VMEM((CHUNK,), jnp.int32),
            pltpu.VMEM((CHUNK,), jnp.int32),
            pltpu.VMEM((CHUNK, feat), jnp.float32),
            pltpu.VMEM((ROWBLK, feat), jnp.float32),
            pltpu.SemaphoreType.DMA,
        ],
    )(h, src, dst)


def _mlp_body(h_ref, a0_ref, a1_ref, w_ref, b_ref, o_ref):
    acc = h_ref[...] + a0_ref[...] + a1_ref[...]
    y = jnp.dot(acc, w_ref[...], preferred_element_type=jnp.float32)
    o_ref[...] = jnp.tanh(y + b_ref[...])


def _mlp_final_body(h_ref, a0_ref, a1_ref, w_ref, b_ref, wo_ref, bo_ref, o_ref):
    acc = h_ref[...] + a0_ref[...] + a1_ref[...]
    y = jnp.dot(acc, w_ref[...], preferred_element_type=jnp.float32)
    t = jnp.tanh(y + b_ref[...])
    o_ref[...] = jnp.dot(t, wo_ref[...], preferred_element_type=jnp.float32) + bo_ref[...]


def _mlp_tc(h, a, w, b, wout=None, bout=None):
    n_nodes, feat = h.shape
    blk = 400
    assert n_nodes % blk == 0
    grid = (n_nodes // blk,)
    row_spec = pl.BlockSpec((blk, feat), lambda i: (i, 0))
    full = lambda shape: pl.BlockSpec(shape, lambda i: (0,) * len(shape))
    args = [h, a[0], a[1], w, b.reshape(1, -1)]
    in_specs = [row_spec, row_spec, row_spec, full(w.shape), full((1, feat))]
    if wout is None:
        body, out_cols = _mlp_body, w.shape[1]
    else:
        body, out_cols = _mlp_final_body, wout.shape[1]
        args += [wout, bout.reshape(1, -1)]
        in_specs += [full(wout.shape), full((1, wout.shape[1]))]
    return pl.pallas_call(
        body,
        grid=grid,
        in_specs=in_specs,
        out_specs=pl.BlockSpec((blk, out_cols), lambda i: (i, 0)),
        out_shape=jax.ShapeDtypeStruct((n_nodes, out_cols), jnp.float32),
    )(*args)


def kernel(x, edge_index, W0, b0, W1, b1, W2, b2, Wout, bout):
    src = edge_index[0]
    dst = edge_index[1]
    h = x
    a = _segment_sum_sc(h, src, dst)
    h = _mlp_tc(h, a, W0, b0)
    a = _segment_sum_sc(h, src, dst)
    h = _mlp_tc(h, a, W1, b1)
    a = _segment_sum_sc(h, src, dst)
    return _mlp_tc(h, a, W2, b2, Wout, bout)


# speedup vs baseline: 8.6689x; 1.7827x over previous
import functools

import jax
import jax.numpy as jnp
from jax import lax
from jax.experimental import pallas as pl
from jax.experimental.pallas import tpu as pltpu
from jax.experimental.pallas import tpu_sc as plsc

NC = 2
NS = 16
NW = NC * NS
CHUNK = 80


ROWBLK = 80
GRP = 1


def _seg_sum_body(n_nodes, feat, edges_per_w, h_hbm, src_hbm, dst_hbm,
                  out_hbm, agg_sh, flat_v, didx2_v, rows_a, rows_b,
                  gsem_a, gsem_b, ssem_a, ssem_b):
    c = lax.axis_index("c")
    s = lax.axis_index("s")
    wid = c * NS + s
    nchunks = edges_per_w // CHUNK
    ngroups = nchunks // GRP
    base = wid * edges_per_w

    nblk = n_nodes // ROWBLK
    blk_iters = (nblk + NS - 1) // NS

    zsrc = rows_a.at[0]

    def zero_row(r, _):
        for j in range(feat // 16):
            zsrc[r, pl.ds(j * 16, 16)] = jnp.zeros((16,), jnp.float32)
        return 0
    lax.fori_loop(0, ROWBLK, zero_row, 0)

    def zero_blk(k, _):
        b = s + k * NS

        @pl.when(b < nblk)
        def _():
            pltpu.sync_copy(zsrc, agg_sh.at[pl.ds(b * ROWBLK, ROWBLK)])
        return 0
    lax.fori_loop(0, blk_iters, zero_blk, 0)

    pltpu.sync_copy(dst_hbm.at[pl.ds(base, edges_per_w)], flat_v)

    def transcribe(r, _):
        for j in range(CHUNK // 16):
            didx2_v[r, pl.ds(j * 16, 16)] = flat_v[pl.ds(r * CHUNK + j * 16, 16)]
        return 0
    lax.fori_loop(0, nchunks, transcribe, 0)
    pltpu.sync_copy(src_hbm.at[pl.ds(base, edges_per_w)], flat_v)

    plsc.subcore_barrier()

    def gather_desc(g, j, rows, sem):
        k = g * GRP + j
        return pltpu.make_async_copy(
            h_hbm.at[flat_v.at[pl.ds(k * CHUNK, CHUNK)]], rows.at[j], sem)

    def scatter_desc(g, j, rows, sem):
        k = g * GRP + j
        return pltpu.make_async_copy(rows.at[j], agg_sh.at[didx2_v.at[k]], sem)

    def fire_gather(g, rows, sem):
        for j in range(GRP):
            k = g * GRP + j
            pltpu.async_copy(
                h_hbm.at[flat_v.at[pl.ds(k * CHUNK, CHUNK)]], rows.at[j], sem)

    def drain_gather(g, rows, sem):
        for j in range(GRP):
            gather_desc(g, j, rows, sem).wait()

    def fire_scatter(g, rows, sem):
        for j in range(GRP):
            k = g * GRP + j
            pltpu.async_copy(rows.at[j], agg_sh.at[didx2_v.at[k]], sem, add=True)

    def drain_scatter(g, rows, sem):
        for j in range(GRP):
            scatter_desc(g, j, rows, sem).wait()

    fire_gather(0, rows_a, gsem_a)

    def pipe(p, _):
        ga = 2 * p
        gb = 2 * p + 1
        drain_gather(ga, rows_a, gsem_a)
        fire_scatter(ga, rows_a, ssem_a)

        @pl.when(p > 0)
        def _():
            drain_scatter(gb - 2, rows_b, ssem_b)
        fire_gather(gb, rows_b, gsem_b)
        drain_gather(gb, rows_b, gsem_b)
        fire_scatter(gb, rows_b, ssem_b)
        drain_scatter(ga, rows_a, ssem_a)
        fire_gather(gb + 1, rows_a, gsem_a)
        return 0
    lax.fori_loop(0, ngroups // 2, pipe, 0)

    glast = ngroups - 1
    drain_gather(glast, rows_a, gsem_a)
    fire_scatter(glast, rows_a, ssem_a)
    drain_scatter(glast - 1, rows_b, ssem_b)
    drain_scatter(glast, rows_a, ssem_a)
    plsc.subcore_barrier()

    def wb_blk(k, _):
        b = s + k * NS

        @pl.when(b < nblk)
        def _():
            pltpu.sync_copy(agg_sh.at[pl.ds(b * ROWBLK, ROWBLK)],
                            out_hbm.at[c, pl.ds(b * ROWBLK, ROWBLK)])
        return 0
    lax.fori_loop(0, blk_iters, wb_blk, 0)


def _segment_sum_sc(h, src, dst):
    n_nodes, feat = h.shape
    e = src.shape[0]
    assert e % NW == 0
    edges_per_w = e // NW
    nchunks = edges_per_w // CHUNK
    assert edges_per_w % CHUNK == 0 and nchunks % GRP == 0
    assert (nchunks // GRP) % 2 == 1 and nchunks // GRP >= 3
    mesh = plsc.VectorSubcoreMesh(core_axis_name="c", subcore_axis_name="s")
    body = functools.partial(_seg_sum_body, n_nodes, feat, edges_per_w)
    return pl.kernel(
        body,
        out_type=jax.ShapeDtypeStruct((NC, n_nodes, feat), jnp.float32),
        mesh=mesh,
        scratch_types=[
            pltpu.VMEM_SHARED((n_nodes, feat), jnp.float32),
            pltpu.VMEM((edges_per_w,), jnp.int32),
            pltpu.VMEM((nchunks, CHUNK), jnp.int32),
            pltpu.VMEM((GRP, CHUNK, feat), jnp.float32),
            pltpu.VMEM((GRP, CHUNK, feat), jnp.float32),
            pltpu.SemaphoreType.DMA,
            pltpu.SemaphoreType.DMA,
            pltpu.SemaphoreType.DMA,
            pltpu.SemaphoreType.DMA,
        ],
    )(h, src, dst)


def _mlp_body(h_ref, a0_ref, a1_ref, w_ref, b_ref, o_ref):
    acc = h_ref[...] + a0_ref[...] + a1_ref[...]
    y = jnp.dot(acc, w_ref[...], preferred_element_type=jnp.float32)
    o_ref[...] = jnp.tanh(y + b_ref[...])


def _mlp_final_body(h_ref, a0_ref, a1_ref, w_ref, b_ref, wo_ref, bo_ref, o_ref):
    acc = h_ref[...] + a0_ref[...] + a1_ref[...]
    y = jnp.dot(acc, w_ref[...], preferred_element_type=jnp.float32)
    t = jnp.tanh(y + b_ref[...])
    o_ref[...] = jnp.dot(t, wo_ref[...], preferred_element_type=jnp.float32) + bo_ref[...]


def _mlp_tc(h, a, w, b, wout=None, bout=None):
    n_nodes, feat = h.shape
    blk = 400
    assert n_nodes % blk == 0
    grid = (n_nodes // blk,)
    row_spec = pl.BlockSpec((blk, feat), lambda i: (i, 0))
    full = lambda shape: pl.BlockSpec(shape, lambda i: (0,) * len(shape))
    args = [h, a[0], a[1], w, b.reshape(1, -1)]
    in_specs = [row_spec, row_spec, row_spec, full(w.shape), full((1, feat))]
    if wout is None:
        body, out_cols = _mlp_body, w.shape[1]
    else:
        body, out_cols = _mlp_final_body, wout.shape[1]
        args += [wout, bout.reshape(1, -1)]
        in_specs += [full(wout.shape), full((1, wout.shape[1]))]
    return pl.pallas_call(
        body,
        grid=grid,
        in_specs=in_specs,
        out_specs=pl.BlockSpec((blk, out_cols), lambda i: (i, 0)),
        out_shape=jax.ShapeDtypeStruct((n_nodes, out_cols), jnp.float32),
    )(*args)


def kernel(x, edge_index, W0, b0, W1, b1, W2, b2, Wout, bout):
    src = edge_index[0]
    dst = edge_index[1]
    h = x
    a = _segment_sum_sc(h, src, dst)
    h = _mlp_tc(h, a, W0, b0)
    a = _segment_sum_sc(h, src, dst)
    h = _mlp_tc(h, a, W1, b1)
    a = _segment_sum_sc(h, src, dst)
    return _mlp_tc(h, a, W2, b2, Wout, bout)


# speedup vs baseline: 12.4550x; 1.4367x over previous
import functools

import jax
import jax.numpy as jnp
from jax import lax
from jax.experimental import pallas as pl
from jax.experimental.pallas import tpu as pltpu
from jax.experimental.pallas import tpu_sc as plsc

NC = 2
NS = 16
NW = NC * NS
CHUNK = 80


ROWBLK = 80
GRP = 1


def _seg_sum_body(n_nodes, feat, edges_per_w, h_hbm, src_hbm, dst_hbm,
                  out_hbm, agg_sh, flat_v, didx_v, rows_v,
                  gsem_a, gsem_b, gsem_c, ssem_a, ssem_b, ssem_c):
    c = lax.axis_index("c")
    s = lax.axis_index("s")
    wid = c * NS + s
    nchunks = edges_per_w // CHUNK
    ngroups = nchunks // GRP
    base = wid * edges_per_w

    nblk = n_nodes // ROWBLK
    blk_iters = (nblk + NS - 1) // NS

    zsrc = rows_v.at[0]

    def zero_row(r, _):
        for j in range(feat // 16):
            zsrc[r, pl.ds(j * 16, 16)] = jnp.zeros((16,), jnp.float32)
        return 0
    lax.fori_loop(0, ROWBLK, zero_row, 0)

    def zero_blk(k, _):
        b = s + k * NS

        @pl.when(b < nblk)
        def _():
            pltpu.sync_copy(zsrc, agg_sh.at[pl.ds(b * ROWBLK, ROWBLK)])
        return 0
    lax.fori_loop(0, blk_iters, zero_blk, 0)

    pltpu.sync_copy(dst_hbm.at[pl.ds(base, edges_per_w)], didx_v)
    pltpu.sync_copy(src_hbm.at[pl.ds(base, edges_per_w)], flat_v)

    plsc.subcore_barrier()

    gsems = (gsem_a, gsem_b, gsem_c)
    ssems = (ssem_a, ssem_b, ssem_c)

    def fire_gather(k, m):
        pltpu.async_copy(
            h_hbm.at[flat_v.at[pl.ds(k * CHUNK, CHUNK)]], rows_v.at[m], gsems[m])

    def drain_gather(k, m):
        pltpu.make_async_copy(
            h_hbm.at[flat_v.at[pl.ds(k * CHUNK, CHUNK)]], rows_v.at[m],
            gsems[m]).wait()

    def fire_scatter(k, m):
        pltpu.async_copy(rows_v.at[m],
                         agg_sh.at[didx_v.at[pl.ds(k * CHUNK, CHUNK)]],
                         ssems[m], add=True)

    def drain_scatter(k, m):
        pltpu.make_async_copy(rows_v.at[m],
                              agg_sh.at[didx_v.at[pl.ds(k * CHUNK, CHUNK)]],
                              ssems[m]).wait()

    fire_gather(0, 0)
    fire_gather(1, 1)

    def pipe(i, _):
        for m in range(3):
            k = 3 * i + m
            drain_gather(k, m)
            fire_scatter(k, m)
            mn = (m + 2) % 3

            @pl.when(k > 0)
            def _():
                drain_scatter(k - 1, mn)
            fire_gather(k + 2, mn)
        return 0
    nfull = (nchunks - 2) // 3
    lax.fori_loop(0, nfull, pipe, 0)
    for k in range(3 * nfull, nchunks):
        m = k % 3
        drain_gather(k, m)
        fire_scatter(k, m)
        drain_scatter(k - 1, (k + 2) % 3)
    drain_scatter(nchunks - 1, (nchunks - 1) % 3)
    plsc.subcore_barrier()

    def wb_blk(k, _):
        b = s + k * NS

        @pl.when(b < nblk)
        def _():
            pltpu.sync_copy(agg_sh.at[pl.ds(b * ROWBLK, ROWBLK)],
                            out_hbm.at[c, pl.ds(b * ROWBLK, ROWBLK)])
        return 0
    lax.fori_loop(0, blk_iters, wb_blk, 0)


def _segment_sum_sc(h, src, dst):
    n_nodes, feat = h.shape
    e = src.shape[0]
    assert e % NW == 0
    edges_per_w = e // NW
    nchunks = edges_per_w // CHUNK
    assert edges_per_w % CHUNK == 0 and nchunks >= 5
    mesh = plsc.VectorSubcoreMesh(core_axis_name="c", subcore_axis_name="s")
    body = functools.partial(_seg_sum_body, n_nodes, feat, edges_per_w)
    return pl.kernel(
        body,
        out_type=jax.ShapeDtypeStruct((NC, n_nodes, feat), jnp.float32),
        mesh=mesh,
        scratch_types=[
            pltpu.VMEM_SHARED((n_nodes, feat), jnp.float32),
            pltpu.VMEM((edges_per_w,), jnp.int32),
            pltpu.VMEM((edges_per_w,), jnp.int32),
            pltpu.VMEM((3, CHUNK, feat), jnp.float32),
            pltpu.SemaphoreType.DMA,
            pltpu.SemaphoreType.DMA,
            pltpu.SemaphoreType.DMA,
            pltpu.SemaphoreType.DMA,
            pltpu.SemaphoreType.DMA,
            pltpu.SemaphoreType.DMA,
        ],
    )(h, src, dst)


def _mlp_body(h_ref, a0_ref, a1_ref, w_ref, b_ref, o_ref):
    acc = h_ref[...] + a0_ref[...] + a1_ref[...]
    y = jnp.dot(acc, w_ref[...], preferred_element_type=jnp.float32)
    o_ref[...] = jnp.tanh(y + b_ref[...])


def _mlp_final_body(h_ref, a0_ref, a1_ref, w_ref, b_ref, wo_ref, bo_ref, o_ref):
    acc = h_ref[...] + a0_ref[...] + a1_ref[...]
    y = jnp.dot(acc, w_ref[...], preferred_element_type=jnp.float32)
    t = jnp.tanh(y + b_ref[...])
    o_ref[...] = jnp.dot(t, wo_ref[...], preferred_element_type=jnp.float32) + bo_ref[...]


def _mlp_tc(h, a, w, b, wout=None, bout=None):
    n_nodes, feat = h.shape
    blk = 400
    assert n_nodes % blk == 0
    grid = (n_nodes // blk,)
    row_spec = pl.BlockSpec((blk, feat), lambda i: (i, 0))
    full = lambda shape: pl.BlockSpec(shape, lambda i: (0,) * len(shape))
    args = [h, a[0], a[1], w, b.reshape(1, -1)]
    in_specs = [row_spec, row_spec, row_spec, full(w.shape), full((1, feat))]
    if wout is None:
        body, out_cols = _mlp_body, w.shape[1]
    else:
        body, out_cols = _mlp_final_body, wout.shape[1]
        args += [wout, bout.reshape(1, -1)]
        in_specs += [full(wout.shape), full((1, wout.shape[1]))]
    return pl.pallas_call(
        body,
        grid=grid,
        in_specs=in_specs,
        out_specs=pl.BlockSpec((blk, out_cols), lambda i: (i, 0)),
        out_shape=jax.ShapeDtypeStruct((n_nodes, out_cols), jnp.float32),
    )(*args)


def kernel(x, edge_index, W0, b0, W1, b1, W2, b2, Wout, bout):
    src = edge_index[0]
    dst = edge_index[1]
    h = x
    a = _segment_sum_sc(h, src, dst)
    h = _mlp_tc(h, a, W0, b0)
    a = _segment_sum_sc(h, src, dst)
    h = _mlp_tc(h, a, W1, b1)
    a = _segment_sum_sc(h, src, dst)
    return _mlp_tc(h, a, W2, b2, Wout, bout)
